# baseline (speedup 1.0000x reference)
import functools

import jax
import jax.numpy as jnp
from jax import lax
from jax.experimental import pallas as pl
from jax.experimental.pallas import tpu as pltpu
from jax.experimental.pallas import tpu_sc as plsc

NC = 2
NS = 16
CH = 128


def _sc_gather_body(nchunks, k_sample, s_base_nodes, s_base_nidx,
                    feat_hbm, nodes_hbm, nidx_hbm, self_out, nsum_out,
                    sidx_all, nidx_all,
                    rows_0, rows_1, acc_0, acc_1, acc_2, acc_3,
                    sem_i, sem_s0, sem_s1,
                    sem_n0, sem_n1, sem_n2, sem_n3):
    assert nchunks == 4
    wid = lax.axis_index("s") * NC + lax.axis_index("c")
    half = nchunks * CH
    w_base = wid * half
    cpi = pltpu.async_copy(
        nidx_hbm.at[pl.ds(0, k_sample), pl.ds(s_base_nidx + w_base, half)],
        nidx_all, sem_i)
    cps = pltpu.async_copy(
        nodes_hbm.at[pl.ds(s_base_nodes + w_base, half)], sidx_all, sem_i)
    cpi.wait()
    cps.wait()

    accs = (acc_0, acc_1, acc_2, acc_3)
    sems_n = (sem_n0, sem_n1, sem_n2, sem_n3)
    rowss = (rows_0, rows_1)
    sems_s = (sem_s0, sem_s1)

    def fire_init(c):
        coff = c * CH
        return (pltpu.async_copy(
                    feat_hbm.at[nidx_all.at[0, pl.ds(coff, CH)]],
                    accs[c], sems_n[c]),
                pltpu.async_copy(
                    feat_hbm.at[sidx_all.at[pl.ds(coff, CH)]],
                    rowss[c % 2], sems_s[c % 2]))

    def fire_adds(c):
        coff = c * CH
        return [pltpu.async_copy(
                    feat_hbm.at[nidx_all.at[k, pl.ds(coff, CH)]],
                    accs[c], sems_n[c], add=True)
                for k in range(1, k_sample)]

    def writeout_self(c):
        pltpu.sync_copy(rowss[c % 2], self_out.at[pl.ds(w_base + c * CH, CH)])

    def writeout_acc(c):
        pltpu.sync_copy(accs[c], nsum_out.at[pl.ds(w_base + c * CH, CH)])

    i0, s0 = fire_init(0)
    i1, s1 = fire_init(1)
    i0.wait(); a0 = fire_adds(0)
    i1.wait(); a1 = fire_adds(1)
    s0.wait(); writeout_self(0)
    i2, s2 = fire_init(2)
    s1.wait(); writeout_self(1)
    i3, s3 = fire_init(3)
    i2.wait(); a2 = fire_adds(2)
    i3.wait(); a3 = fire_adds(3)
    for cp in a0:
        cp.wait()
    writeout_acc(0)
    for cp in a1:
        cp.wait()
    writeout_acc(1)
    s2.wait(); writeout_self(2)
    s3.wait(); writeout_self(3)
    for cp in a2:
        cp.wait()
    writeout_acc(2)
    for cp in a3:
        cp.wait()
    writeout_acc(3)


def _tc_body(s_ref, n_ref, w1_ref, w2_ref, wt_ref, o_ref, *, inv_k):
    h = (jnp.dot(s_ref[...], w1_ref[...])
         + jnp.dot(n_ref[...] * inv_k, w2_ref[...]))
    h = jnp.maximum(h, 0.0)
    o_ref[...] = jnp.dot(h, wt_ref[...])


def kernel(feat, W_agg, weight, nodes, labels, neigh_idx):
    del labels
    B = nodes.shape[0]
    K = neigh_idx.shape[1]
    D = feat.shape[1]
    C = weight.shape[0]
    NW = NC * NS
    NSPLIT = 2
    BS = B // NSPLIT
    assert BS % (NW * CH) == 0
    nchunks = BS // (NW * CH)

    mesh = plsc.VectorSubcoreMesh(
        core_axis_name="c", subcore_axis_name="s",
        num_cores=NC, num_subcores=NS)

    def make_sc(s_base_nodes, s_base_nidx):
        return pl.kernel(
            functools.partial(_sc_gather_body, nchunks, K, s_base_nodes,
                              s_base_nidx),
            out_type=(jax.ShapeDtypeStruct((BS, D), jnp.float32),
                      jax.ShapeDtypeStruct((BS, D), jnp.float32)),
            mesh=mesh,
            scratch_types=(
                [pltpu.VMEM((BS // NW,), jnp.int32),
                 pltpu.VMEM((K, BS // NW), jnp.int32)]
                + [pltpu.VMEM((CH, D), jnp.float32)] * 6
                + [pltpu.SemaphoreType.DMA] * 7),
        )

    CP = 8
    w1 = W_agg[:D]
    w2 = W_agg[D:]
    wt = jnp.zeros((D, CP), jnp.float32).at[:, :C].set(weight.T)
    bm = 4096
    tc_dense = pl.pallas_call(
        functools.partial(_tc_body, inv_k=1.0 / K),
        grid=(BS // bm,),
        in_specs=[
            pl.BlockSpec((bm, D), lambda i: (i, 0)),
            pl.BlockSpec((bm, D), lambda i: (i, 0)),
            pl.BlockSpec((D, D), lambda i: (0, 0)),
            pl.BlockSpec((D, D), lambda i: (0, 0)),
            pl.BlockSpec((D, CP), lambda i: (0, 0)),
        ],
        out_specs=pl.BlockSpec((bm, CP), lambda i: (i, 0)),
        out_shape=jax.ShapeDtypeStruct((BS, CP), jnp.float32),
    )
    nidx_t = neigh_idx.T
    outs = []
    for s in range(NSPLIT):
        self_f, nsum = make_sc(s * BS, s * BS)(feat, nodes, nidx_t)
        outs.append(tc_dense(self_f, nsum, w1, w2, wt))
    return jnp.concatenate(outs, axis=0)[:, :C]
